# baseline (speedup 1.0000x reference)
import functools

import jax
import jax.numpy as jnp
from jax import lax
from jax.experimental import pallas as pl
from jax.experimental.pallas import tpu as pltpu
from jax.experimental.pallas import tpu_sc as plsc

_B, _S, _H, _V = 4, 4096, 768, 100000
_N = _B * _S
_NC, _NS = 2, 16
_NW = _NC * _NS
_TPW = _N // _NW
_K = 8
_NCHUNK = _TPW // _K
_NG = _NCHUNK // 2
_NVJ = _H // 16
_NT = 7


def _ngram_hash_sums(bytes_f32):
    seq_length = bytes_f32.shape[1]
    terms = [bytes_f32[:, k:] * jnp.float32(256.0 ** k) for k in range(8)]

    def padw(x):
        return jnp.pad(x, ((0, 0), (0, seq_length - x.shape[1])))

    t = [padw(x) for x in terms]
    sums = {}
    for n in range(3, 9):
        m = 1
        while m < n:
            m *= 2
        lvl = t[:n] + [None] * (m - n)
        while len(lvl) > 1:
            half = len(lvl) // 2
            nxt = []
            for i in range(half):
                a, b = lvl[i], lvl[i + half]
                nxt.append(a if b is None else (b if a is None else a + b))
            lvl = nxt
        sums[n] = lvl[0]
    return sums


_MODV = 100000
_P64 = 51616
_C64 = 32704
_PW2 = [2, 4, 16, 256, 65536, 67296]


def _modmul(a, b):
    a1 = a // 100
    a0 = a - a1 * 100
    t = ((a1 * b) % _MODV * 100) % _MODV
    return (t + a0 * b) % _MODV


def _mod100k_f32(v):
    bits = lax.bitcast_convert_type(v, jnp.int32)
    e = ((bits >> 23) & 0xFF) - 127
    m24 = (bits & 0x7FFFFF) | (1 << 23)
    shift = e - 23
    m = jnp.where(shift < jnp.int32(0),
                  m24 >> jnp.minimum(-shift, jnp.int32(24)), m24)
    sh = jnp.maximum(shift, jnp.int32(0))
    p = jnp.ones_like(sh)
    for b in range(6):
        hit = ((sh >> b) & 1) == jnp.int32(1)
        p = jnp.where(hit, _modmul(p, jnp.int32(_PW2[b])), p)
    r = _modmul(m % _MODV, p)
    r = jnp.where(e == jnp.int32(63), (r + (_MODV - _P64)) % _MODV, r)
    r = jnp.where(e == jnp.int32(64), jnp.int32(_C64), r)
    r = jnp.where(v == jnp.float32(0.0), jnp.int32(0), r)
    return r


def _sc_lookup_combine(idx, W_byte, W3, W4, W5, W6, W7, W8):
    mesh = plsc.VectorSubcoreMesh(core_axis_name="c", subcore_axis_name="s")

    @functools.partial(
        pl.kernel,
        mesh=mesh,
        out_type=jax.ShapeDtypeStruct((_N, _H), jnp.float32),
        scratch_types=(
            [pltpu.VMEM((_NCHUNK * _NT * _K,), jnp.int32)]
            + [pltpu.VMEM((_K, _H), jnp.float32) for _ in range(2 * _NT)]
            + [pltpu.SemaphoreType.DMA, pltpu.SemaphoreType.DMA,
               pltpu.SemaphoreType.DMA]
        ),
    )
    def run(idx_hbm, wb, w3, w4, w5, w6, w7, w8, out_hbm,
            idxv,
            a0, a1, a2, a3, a4, a5, a6,
            c0, c1, c2, c3, c4, c5, c6,
            sema, semb, semo):
        tables = (wb, w3, w4, w5, w6, w7, w8)
        bufsA = (a0, a1, a2, a3, a4, a5, a6)
        bufsB = (c0, c1, c2, c3, c4, c5, c6)
        wid = lax.axis_index("s") * jnp.int32(_NC) + lax.axis_index("c")
        base = wid * jnp.int32(_TPW)
        pltpu.sync_copy(idx_hbm.at[wid], idxv)

        def fire(ci, bufs, sem, ts=range(_NT)):
            for t in ts:
                flat = (ci * jnp.int32(_NT) + jnp.int32(t)) * jnp.int32(_K)
                pltpu.async_copy(
                    tables[t].at[idxv.at[pl.ds(flat, _K)]], bufs[t], sem)

        def drain(bufs, sem):
            for t in range(_NT):
                pltpu.make_async_copy(
                    tables[t].at[idxv.at[pl.ds(jnp.int32(0), _K)]], bufs[t],
                    sem).wait()

        def compute(cb, bufs):
            def tok_body(i, _):
                pos = lax.rem(cb + i, jnp.int32(_S))
                posv = jnp.full((16,), pos, dtype=jnp.int32)
                scales = [
                    jnp.where(posv < (_S - n + 1),
                              jnp.float32(1.0 / n), jnp.float32(0.0))
                    for n in range(3, 9)
                ]
                for j in range(_NVJ):
                    sl = pl.ds(j * 16, 16)
                    acc = bufs[0][i, sl]
                    for t in range(6):
                        acc = acc + scales[t] * bufs[t + 1][i, sl]
                    bufs[0][i, sl] = acc
                return _

            lax.fori_loop(jnp.int32(0), jnp.int32(_K), tok_body, None)

        fire(jnp.int32(0), bufsA, sema)

        def pair_body(g, carry):
            ca = g * jnp.int32(2)
            cb_ = ca + jnp.int32(1)
            fire(cb_, bufsB, semb)
            drain(bufsA, sema)
            compute(base + ca * jnp.int32(_K), bufsA)
            cpa = pltpu.async_copy(
                bufsA[0], out_hbm.at[pl.ds(base + ca * jnp.int32(_K), _K)],
                semo)
            nxt = jnp.minimum(ca + jnp.int32(2), jnp.int32(_NCHUNK - 1))
            fire(nxt, bufsA, sema, ts=range(1, _NT))
            cpa.wait()
            fire(nxt, bufsA, sema, ts=range(0, 1))
            drain(bufsB, semb)
            compute(base + cb_ * jnp.int32(_K), bufsB)
            cpb = pltpu.async_copy(
                bufsB[0], out_hbm.at[pl.ds(base + cb_ * jnp.int32(_K), _K)],
                semo)
            cpb.wait()
            return carry

        lax.fori_loop(jnp.int32(0), jnp.int32(_NG), pair_body, None)
        drain(bufsA, sema)

    return run(idx, W_byte, W3, W4, W5, W6, W7, W8)


def kernel(bytes_input, W_byte, W_ng0, W_ng1, W_ng2, W_ng3, W_ng4, W_ng5):
    tables = [W_ng0, W_ng1, W_ng2, W_ng3, W_ng4, W_ng5]
    b32 = bytes_input.astype(jnp.int32)
    bf = b32.astype(jnp.float32)
    sums = _ngram_hash_sums(bf)
    idx_list = [b32.reshape(_N)]
    for n in range(3, 9):
        hv = sums[n][:, :_S - n + 1]
        h = _mod100k_f32(hv)
        h = jnp.pad(h, ((0, 0), (0, n - 1)))
        idx_list.append(h.reshape(_N))
    idx = jnp.stack(idx_list)
    idx = (idx.reshape(_NT, _NW, _NCHUNK, _K)
              .transpose(1, 2, 0, 3)
              .reshape(_NW, _NCHUNK * _NT * _K))
    out = _sc_lookup_combine(idx, W_byte, *tables)
    return out.reshape(_B, _S, _H)


# speedup vs baseline: 1.1601x; 1.1601x over previous
import functools

import jax
import jax.numpy as jnp
from jax import lax
from jax.experimental import pallas as pl
from jax.experimental.pallas import tpu as pltpu
from jax.experimental.pallas import tpu_sc as plsc

_B, _S, _H, _V = 4, 4096, 768, 100000
_N = _B * _S
_NC, _NS = 2, 16
_NW = _NC * _NS
_TPW = _N // _NW
_K = 8
_NCHUNK = _TPW // _K
_NG = _NCHUNK // 2
_NVJ = _H // 16
_NT = 7


def _ngram_hash_sums(bytes_f32):
    seq_length = bytes_f32.shape[1]
    terms = [bytes_f32[:, k:] * jnp.float32(256.0 ** k) for k in range(8)]

    def padw(x):
        return jnp.pad(x, ((0, 0), (0, seq_length - x.shape[1])))

    t = [padw(x) for x in terms]
    sums = {}
    for n in range(3, 9):
        m = 1
        while m < n:
            m *= 2
        lvl = t[:n] + [None] * (m - n)
        while len(lvl) > 1:
            half = len(lvl) // 2
            nxt = []
            for i in range(half):
                a, b = lvl[i], lvl[i + half]
                nxt.append(a if b is None else (b if a is None else a + b))
            lvl = nxt
        sums[n] = lvl[0]
    return sums


def _sc_lookup_combine(idx, W_byte, W3, W4, W5, W6, W7, W8):
    mesh = plsc.VectorSubcoreMesh(core_axis_name="c", subcore_axis_name="s")

    @functools.partial(
        pl.kernel,
        mesh=mesh,
        out_type=jax.ShapeDtypeStruct((_N, _H), jnp.float32),
        scratch_types=(
            [pltpu.VMEM((_NCHUNK * _NT * _K,), jnp.int32)]
            + [pltpu.VMEM((_K, _H), jnp.float32) for _ in range(2 * _NT)]
            + [pltpu.VMEM_SHARED((256, _H), jnp.float32)]
            + [pltpu.SemaphoreType.DMA] * 5
        ),
    )
    def run(idx_hbm, wb, w3, w4, w5, w6, w7, w8, out_hbm,
            idxv,
            a0, a1, a2, a3, a4, a5, a6,
            c0, c1, c2, c3, c4, c5, c6,
            wbs, sema, semb, semo, semwa, semwb):
        @pl.when(lax.axis_index("s") == jnp.int32(0))
        def _stage():
            pltpu.sync_copy(wb, wbs)

        plsc.subcore_barrier()
        tables = (wb, w3, w4, w5, w6, w7, w8)
        bufsA = (a0, a1, a2, a3, a4, a5, a6)
        bufsB = (c0, c1, c2, c3, c4, c5, c6)
        wid = lax.axis_index("s") * jnp.int32(_NC) + lax.axis_index("c")
        base = wid * jnp.int32(_TPW)
        pltpu.sync_copy(idx_hbm.at[wid], idxv)

        def fire(ci, bufs, sem):
            for t in range(1, _NT):
                flat = (ci * jnp.int32(_NT) + jnp.int32(t)) * jnp.int32(_K)
                pltpu.async_copy(
                    tables[t].at[idxv.at[pl.ds(flat, _K)]], bufs[t], sem)

        def fire_byte(ci, bufs, semw):
            bvec = idxv[pl.ds(ci * jnp.int32(_NT * _K), 16)]
            for i in range(_K):
                pltpu.async_copy(wbs.at[bvec[i]], bufs[0].at[jnp.int32(i)],
                                 semw)

        def drain(bufs, sem, semw):
            for t in range(1, _NT):
                pltpu.make_async_copy(
                    tables[t].at[idxv.at[pl.ds(jnp.int32(0), _K)]], bufs[t],
                    sem).wait()
            for i in range(_K):
                pltpu.make_async_copy(
                    wbs.at[jnp.int32(0)], bufs[0].at[jnp.int32(i)],
                    semw).wait()

        def compute(cb, bufs):
            def tok_body(i, _):
                pos = lax.rem(cb + i, jnp.int32(_S))
                posv = jnp.full((16,), pos, dtype=jnp.int32)
                scales = [
                    jnp.where(posv < (_S - n + 1),
                              jnp.float32(1.0 / n), jnp.float32(0.0))
                    for n in range(3, 9)
                ]
                for j in range(_NVJ):
                    sl = pl.ds(j * 16, 16)
                    acc = bufs[0][i, sl]
                    for t in range(6):
                        acc = acc + scales[t] * bufs[t + 1][i, sl]
                    bufs[0][i, sl] = acc
                return _

            lax.fori_loop(jnp.int32(0), jnp.int32(_K), tok_body, None)

        fire(jnp.int32(0), bufsA, sema)
        fire_byte(jnp.int32(0), bufsA, semwa)

        def pair_body(g, carry):
            ca = g * jnp.int32(2)
            cb_ = ca + jnp.int32(1)
            fire(cb_, bufsB, semb)
            fire_byte(cb_, bufsB, semwb)
            drain(bufsA, sema, semwa)
            compute(base + ca * jnp.int32(_K), bufsA)
            cpa = pltpu.async_copy(
                bufsA[0], out_hbm.at[pl.ds(base + ca * jnp.int32(_K), _K)],
                semo)
            nxt = jnp.minimum(ca + jnp.int32(2), jnp.int32(_NCHUNK - 1))
            fire(nxt, bufsA, sema)
            cpa.wait()
            fire_byte(nxt, bufsA, semwa)
            drain(bufsB, semb, semwb)
            compute(base + cb_ * jnp.int32(_K), bufsB)
            cpb = pltpu.async_copy(
                bufsB[0], out_hbm.at[pl.ds(base + cb_ * jnp.int32(_K), _K)],
                semo)
            cpb.wait()
            return carry

        lax.fori_loop(jnp.int32(0), jnp.int32(_NG), pair_body, None)
        drain(bufsA, sema, semwa)

    return run(idx, W_byte, W3, W4, W5, W6, W7, W8)


def kernel(bytes_input, W_byte, W_ng0, W_ng1, W_ng2, W_ng3, W_ng4, W_ng5):
    tables = [W_ng0, W_ng1, W_ng2, W_ng3, W_ng4, W_ng5]
    b32 = bytes_input.astype(jnp.int32)
    bf = b32.astype(jnp.float32)
    sums = _ngram_hash_sums(bf)
    idx_list = [b32.reshape(_N)]
    for n in range(3, 9):
        hv = sums[n][:, :_S - n + 1]
        h = jnp.mod(hv.astype(jnp.int64), tables[n - 3].shape[0])
        h = jnp.pad(h, ((0, 0), (0, n - 1)))
        idx_list.append(h.reshape(_N).astype(jnp.int32))
    idx = jnp.stack(idx_list)
    idx = (idx.reshape(_NT, _NW, _NCHUNK, _K)
              .transpose(1, 2, 0, 3)
              .reshape(_NW, _NCHUNK * _NT * _K))
    out = _sc_lookup_combine(idx, W_byte, *tables)
    return out.reshape(_B, _S, _H)


# speedup vs baseline: 1.1929x; 1.0283x over previous
import functools

import jax
import jax.numpy as jnp
from jax import lax
from jax.experimental import pallas as pl
from jax.experimental.pallas import tpu as pltpu
from jax.experimental.pallas import tpu_sc as plsc

_B, _S, _H, _V = 4, 4096, 768, 100000
_N = _B * _S
_NC, _NS = 2, 16
_NW = _NC * _NS
_TPW = _N // _NW
_K = 8
_NCHUNK = _TPW // _K
_NG = _NCHUNK // 2
_NVJ = _H // 16
_NT = 7


def _ngram_hash_sums(bytes_f32):
    seq_length = bytes_f32.shape[1]
    terms = [bytes_f32[:, k:] * jnp.float32(256.0 ** k) for k in range(8)]

    def padw(x):
        return jnp.pad(x, ((0, 0), (0, seq_length - x.shape[1])))

    t = [padw(x) for x in terms]
    sums = {}
    for n in range(3, 9):
        m = 1
        while m < n:
            m *= 2
        lvl = t[:n] + [None] * (m - n)
        while len(lvl) > 1:
            half = len(lvl) // 2
            nxt = []
            for i in range(half):
                a, b = lvl[i], lvl[i + half]
                nxt.append(a if b is None else (b if a is None else a + b))
            lvl = nxt
        sums[n] = lvl[0]
    return sums


def _sc_lookup_combine(idx, W_byte, W3, W4, W5, W6, W7, W8):
    mesh = plsc.VectorSubcoreMesh(core_axis_name="c", subcore_axis_name="s")

    @functools.partial(
        pl.kernel,
        mesh=mesh,
        out_type=jax.ShapeDtypeStruct((_N, _H), jnp.float32),
        scratch_types=(
            [pltpu.VMEM((_NCHUNK * _NT * _K,), jnp.int32)]
            + [pltpu.VMEM((_K, _H), jnp.float32) for _ in range(2 * _NT)]
            + [pltpu.SemaphoreType.DMA, pltpu.SemaphoreType.DMA,
               pltpu.SemaphoreType.DMA]
        ),
    )
    def run(idx_hbm, wb, w3, w4, w5, w6, w7, w8, out_hbm,
            idxv,
            a0, a1, a2, a3, a4, a5, a6,
            c0, c1, c2, c3, c4, c5, c6,
            sema, semb, semo):
        tables = (wb, w3, w4, w5, w6, w7, w8)
        bufsA = (a0, a1, a2, a3, a4, a5, a6)
        bufsB = (c0, c1, c2, c3, c4, c5, c6)
        wid = lax.axis_index("s") * jnp.int32(_NC) + lax.axis_index("c")
        base = wid * jnp.int32(_TPW)
        pltpu.sync_copy(idx_hbm.at[wid], idxv)

        def fire(ci, bufs, sem, ts=range(_NT)):
            for t in ts:
                flat = (ci * jnp.int32(_NT) + jnp.int32(t)) * jnp.int32(_K)
                pltpu.async_copy(
                    tables[t].at[idxv.at[pl.ds(flat, _K)]], bufs[t], sem)

        def drain(bufs, sem):
            for t in range(_NT):
                pltpu.make_async_copy(
                    tables[t].at[idxv.at[pl.ds(jnp.int32(0), _K)]], bufs[t],
                    sem).wait()

        def compute(cb, bufs):
            def tok_body(i, _):
                pos = lax.rem(cb + i, jnp.int32(_S))
                posv = jnp.full((16,), pos, dtype=jnp.int32)
                scales = [
                    jnp.where(posv < (_S - n + 1),
                              jnp.float32(1.0 / n), jnp.float32(0.0))
                    for n in range(3, 9)
                ]
                for j in range(_NVJ):
                    sl = pl.ds(j * 16, 16)
                    acc = bufs[0][i, sl]
                    for t in range(6):
                        acc = acc + scales[t] * bufs[t + 1][i, sl]
                    bufs[0][i, sl] = acc
                return _

            lax.fori_loop(jnp.int32(0), jnp.int32(_K), tok_body, None)

        fire(jnp.int32(0), bufsA, sema)

        def pair_body(g, carry):
            ca = g * jnp.int32(2)
            cb_ = ca + jnp.int32(1)
            fire(cb_, bufsB, semb)
            drain(bufsA, sema)
            compute(base + ca * jnp.int32(_K), bufsA)
            cpa = pltpu.async_copy(
                bufsA[0], out_hbm.at[pl.ds(base + ca * jnp.int32(_K), _K)],
                semo)
            nxt = jnp.minimum(ca + jnp.int32(2), jnp.int32(_NCHUNK - 1))
            fire(nxt, bufsA, sema, ts=range(1, _NT))
            cpa.wait()
            fire(nxt, bufsA, sema, ts=range(0, 1))
            drain(bufsB, semb)
            compute(base + cb_ * jnp.int32(_K), bufsB)
            cpb = pltpu.async_copy(
                bufsB[0], out_hbm.at[pl.ds(base + cb_ * jnp.int32(_K), _K)],
                semo)
            cpb.wait()
            return carry

        lax.fori_loop(jnp.int32(0), jnp.int32(_NG), pair_body, None)
        drain(bufsA, sema)

    return run(idx, W_byte, W3, W4, W5, W6, W7, W8)


def kernel(bytes_input, W_byte, W_ng0, W_ng1, W_ng2, W_ng3, W_ng4, W_ng5):
    tables = [W_ng0, W_ng1, W_ng2, W_ng3, W_ng4, W_ng5]
    b32 = bytes_input.astype(jnp.int32)
    bf = b32.astype(jnp.float32)
    sums = _ngram_hash_sums(bf)
    idx_list = [b32.reshape(_N)]
    for n in range(3, 9):
        hv = sums[n][:, :_S - n + 1]
        h = jnp.mod(hv.astype(jnp.int64), tables[n - 3].shape[0])
        h = jnp.pad(h, ((0, 0), (0, n - 1)))
        idx_list.append(h.reshape(_N).astype(jnp.int32))
    idx = jnp.stack(idx_list)
    idx = (idx.reshape(_NT, _NW, _NCHUNK, _K)
              .transpose(1, 2, 0, 3)
              .reshape(_NW, _NCHUNK * _NT * _K))
    out = _sc_lookup_combine(idx, W_byte, *tables)
    return out.reshape(_B, _S, _H)
